# baseline (speedup 1.0000x reference)
import functools

import jax
import jax.numpy as jnp
from jax import lax
from jax.experimental import pallas as pl
from jax.experimental.pallas import tpu as pltpu
from jax.experimental.pallas import tpu_sc as plsc

H = 1024
E = 8
I = 1408
S = 2816
N = 2048
BT = 256
GT = 128
NTILES = N * 2 // GT + (E - 1)
NS = NTILES * GT
NC = 2
NSUB = 16
NW = NC * NSUB
BPW = N // NW


def _shift0(a, k):
    return jnp.concatenate([jnp.zeros((k, a.shape[1]), a.dtype), a[:-k]], axis=0)


def _shift1(a, k):
    return jnp.concatenate([jnp.zeros((a.shape[0], k), a.dtype), a[:, :-k]], axis=1)


def _router_body(x_ref, gw_ref, w_ref, dA_ref, dB_ref, tm_ref):
    x = x_ref[...]
    logits = lax.dot_general(x, gw_ref[...], (((1,), (1,)), ((), ())),
                             preferred_element_type=jnp.float32)
    mx = jnp.max(logits, axis=-1, keepdims=True)
    ex = jnp.exp(logits - mx)
    scores = ex / jnp.sum(ex, axis=-1, keepdims=True)
    iota = lax.broadcasted_iota(jnp.int32, scores.shape, 1)
    m1 = jnp.max(scores, axis=-1, keepdims=True)
    i1 = jnp.min(jnp.where(scores == m1, iota, E), axis=-1, keepdims=True)
    sel1 = iota == i1
    masked = jnp.where(sel1, -jnp.inf, scores)
    m2 = jnp.max(masked, axis=-1, keepdims=True)
    i2 = jnp.min(jnp.where(masked == m2, iota, E), axis=-1, keepdims=True)
    sel2 = iota == i2
    maskf = jnp.where(sel1 | sel2, 1.0, 0.0).astype(jnp.float32)

    acc = _shift0(maskf, 1)
    k = 1
    while k < N:
        acc = acc + _shift0(acc, k)
        k *= 2
    rank = acc
    counts = rank[N - 1:N, :] + maskf[N - 1:N, :]
    tiles = jnp.floor((counts + (GT - 1)) / GT)
    texc = _shift1(tiles, 1)
    texc = texc + _shift1(texc, 1)
    texc = texc + _shift1(texc, 2)
    texc = texc + _shift1(texc, 4)
    seg_off = GT * texc
    cum_incl = texc + tiles

    pos = seg_off + rank
    dA_ref[...] = jnp.sum(jnp.where(sel1, pos, 0.0), axis=1,
                          keepdims=True).astype(jnp.int32)
    dB_ref[...] = jnp.sum(jnp.where(sel2, pos, 0.0), axis=1,
                          keepdims=True).astype(jnp.int32)
    w_ref[...] = jnp.concatenate([m1, m2], axis=1)

    tio = lax.broadcasted_iota(jnp.int32, (1, 64), 1).astype(jnp.float32)
    tm = jnp.zeros((1, 64), jnp.float32)
    for e in range(E):
        tm = tm + jnp.where(tio >= cum_incl[:, e:e + 1], 1.0, 0.0)
    tm_ref[...] = jnp.minimum(tm, float(E - 1)).astype(jnp.int32)


def _gffn_body(tm_ref, xs_ref, gu_ref, dn_ref, ys_ref):
    xb = xs_ref[...]
    gu = lax.dot_general(xb, gu_ref[0], (((1,), (1,)), ((), ())),
                         preferred_element_type=jnp.float32)
    g = gu[:, :I]
    u = gu[:, I:]
    h = (g * jax.nn.sigmoid(g) * u).astype(jnp.bfloat16)
    ys_ref[...] = lax.dot_general(h, dn_ref[0], (((1,), (1,)), ((), ())),
                                  preferred_element_type=jnp.float32)


def _shared_body(x_ref, sg_ref, su_ref, sd_ref, seg_ref, out_ref):
    xb = x_ref[...]
    g = lax.dot_general(xb, sg_ref[...], (((1,), (1,)), ((), ())),
                        preferred_element_type=jnp.float32)
    u = lax.dot_general(xb, su_ref[...], (((1,), (1,)), ((), ())),
                        preferred_element_type=jnp.float32)
    h = (g * jax.nn.sigmoid(g) * u).astype(jnp.bfloat16)
    sh = lax.dot_general(h, sd_ref[...], (((1,), (1,)), ((), ())),
                         preferred_element_type=jnp.float32)
    gate_logit = jnp.sum(xb.astype(jnp.float32) * seg_ref[...],
                         axis=1, keepdims=True)
    out_ref[...] = jax.nn.sigmoid(gate_logit) * sh


def _combine_body(y0_ref, y1_ref, w_ref, sh_ref, out_ref):
    w = w_ref[...]
    out_ref[...] = (w[:, 0:1] * y0_ref[...] + w[:, 1:2] * y1_ref[...]
                    + sh_ref[...])


@functools.cache
def _sc_mesh():
    return plsc.VectorSubcoreMesh(core_axis_name="c", subcore_axis_name="s",
                                  num_cores=NC, num_subcores=NSUB)


def _dispatch_tokens(x32, dA, dB):
    @functools.partial(
        pl.kernel,
        out_type=jax.ShapeDtypeStruct((NS, H // 2), jnp.int32),
        mesh=_sc_mesh(),
        scratch_types=[
            pltpu.VMEM((BPW,), jnp.int32),
            pltpu.VMEM((BPW,), jnp.int32),
            pltpu.VMEM((BPW, H // 2), jnp.int32),
        ],
    )
    def scatter_kernel(x_hbm, dA_hbm, dB_hbm, xs_hbm, idxA_v, idxB_v, rows_v):
        wid = lax.axis_index("s") * NC + lax.axis_index("c")
        base = wid * BPW
        pltpu.sync_copy(dA_hbm.at[pl.ds(base, BPW)], idxA_v)
        pltpu.sync_copy(dB_hbm.at[pl.ds(base, BPW)], idxB_v)
        pltpu.sync_copy(x_hbm.at[pl.ds(base, BPW)], rows_v)
        pltpu.sync_copy(rows_v, xs_hbm.at[idxA_v])
        pltpu.sync_copy(rows_v, xs_hbm.at[idxB_v])

    return scatter_kernel(x32, dA, dB)


def _collect_rows(ys, dA, dB):
    @functools.partial(
        pl.kernel,
        out_type=(jax.ShapeDtypeStruct((N, H), jnp.float32),
                  jax.ShapeDtypeStruct((N, H), jnp.float32)),
        mesh=_sc_mesh(),
        scratch_types=[
            pltpu.VMEM((BPW,), jnp.int32),
            pltpu.VMEM((BPW, H), jnp.float32),
            pltpu.SemaphoreType.DMA,
        ],
    )
    def gather_kernel(ys_hbm, dA_hbm, dB_hbm, y0_hbm, y1_hbm, idx_v, rows_v, sem):
        wid = lax.axis_index("s") * NC + lax.axis_index("c")
        base = wid * BPW
        pltpu.sync_copy(dA_hbm.at[pl.ds(base, BPW)], idx_v)
        pltpu.async_copy(ys_hbm.at[idx_v], rows_v, sem).wait()
        pltpu.sync_copy(rows_v, y0_hbm.at[pl.ds(base, BPW)])
        pltpu.sync_copy(dB_hbm.at[pl.ds(base, BPW)], idx_v)
        pltpu.async_copy(ys_hbm.at[idx_v], rows_v, sem).wait()
        pltpu.sync_copy(rows_v, y1_hbm.at[pl.ds(base, BPW)])

    return gather_kernel(ys, dA, dB)


def kernel(x, gate_w, experts_gate_up, experts_down, shared_gate_w,
           shared_up_w, shared_down_w, shared_expert_gate_w):
    Bx, Tx, Hx = x.shape
    xf = x.reshape(Bx * Tx, Hx)
    xbf = xf.astype(jnp.bfloat16)
    gu_bf = experts_gate_up.astype(jnp.bfloat16)
    dn_bf = experts_down.astype(jnp.bfloat16)
    sg_bf = shared_gate_w.astype(jnp.bfloat16)
    su_bf = shared_up_w.astype(jnp.bfloat16)
    sd_bf = shared_down_w.astype(jnp.bfloat16)

    w2, dA2, dB2, tm = pl.pallas_call(
        _router_body,
        grid=(1,),
        in_specs=[
            pl.BlockSpec((N, H), lambda i: (0, 0)),
            pl.BlockSpec((E, H), lambda i: (0, 0)),
        ],
        out_specs=[
            pl.BlockSpec((N, 2), lambda i: (0, 0)),
            pl.BlockSpec((N, 1), lambda i: (0, 0)),
            pl.BlockSpec((N, 1), lambda i: (0, 0)),
            pl.BlockSpec((1, 64), lambda i: (0, 0)),
        ],
        out_shape=[
            jax.ShapeDtypeStruct((N, 2), jnp.float32),
            jax.ShapeDtypeStruct((N, 1), jnp.int32),
            jax.ShapeDtypeStruct((N, 1), jnp.int32),
            jax.ShapeDtypeStruct((1, 64), jnp.int32),
        ],
    )(xf, gate_w)
    dA = dA2.reshape(N)
    dB = dB2.reshape(N)
    tile_map = tm.reshape(64)[:NTILES]

    sh = pl.pallas_call(
        _shared_body,
        grid=(N // BT,),
        in_specs=[
            pl.BlockSpec((BT, H), lambda t: (t, 0)),
            pl.BlockSpec((S, H), lambda t: (0, 0)),
            pl.BlockSpec((S, H), lambda t: (0, 0)),
            pl.BlockSpec((H, S), lambda t: (0, 0)),
            pl.BlockSpec((1, H), lambda t: (0, 0)),
        ],
        out_specs=pl.BlockSpec((BT, H), lambda t: (t, 0)),
        out_shape=jax.ShapeDtypeStruct((N, H), jnp.float32),
    )(xbf, sg_bf, su_bf, sd_bf, shared_expert_gate_w)

    x32 = lax.bitcast_convert_type(xbf.reshape(N, H // 2, 2), jnp.int32)
    xs32 = _dispatch_tokens(x32, dA, dB)
    xs = lax.bitcast_convert_type(xs32, jnp.bfloat16).reshape(NS, H)

    ys = pl.pallas_call(
        _gffn_body,
        grid_spec=pltpu.PrefetchScalarGridSpec(
            num_scalar_prefetch=1,
            grid=(NTILES,),
            in_specs=[
                pl.BlockSpec((GT, H), lambda t, tm_ref: (t, 0)),
                pl.BlockSpec((1, 2 * I, H), lambda t, tm_ref: (tm_ref[t], 0, 0)),
                pl.BlockSpec((1, H, I), lambda t, tm_ref: (tm_ref[t], 0, 0)),
            ],
            out_specs=pl.BlockSpec((GT, H), lambda t, tm_ref: (t, 0)),
        ),
        out_shape=jax.ShapeDtypeStruct((NS, H), jnp.float32),
    )(tile_map, xs, gu_bf, dn_bf)

    y0, y1 = _collect_rows(ys, dA, dB)

    out = pl.pallas_call(
        _combine_body,
        grid=(N // BT,),
        in_specs=[
            pl.BlockSpec((BT, H), lambda t: (t, 0)),
            pl.BlockSpec((BT, H), lambda t: (t, 0)),
            pl.BlockSpec((BT, 2), lambda t: (t, 0)),
            pl.BlockSpec((BT, H), lambda t: (t, 0)),
        ],
        out_specs=pl.BlockSpec((BT, H), lambda t: (t, 0)),
        out_shape=jax.ShapeDtypeStruct((N, H), jnp.float32),
    )(y0, y1, w2, sh)

    return out.reshape(Bx, Tx, Hx)


# speedup vs baseline: 1.8060x; 1.8060x over previous
import functools

import jax
import jax.numpy as jnp
from jax import lax
from jax.experimental import pallas as pl
from jax.experimental.pallas import tpu as pltpu
from jax.experimental.pallas import tpu_sc as plsc

H = 1024
E = 8
I = 1408
S = 2816
N = 2048
BT = 256
GT = 128
NTILES = N * 2 // GT + (E - 1)
NS = NTILES * GT
NC = 2
NSUB = 16
NW = NC * NSUB
BPW = N // NW


def _shift0(a, k):
    return jnp.concatenate([jnp.zeros((k, a.shape[1]), a.dtype), a[:-k]], axis=0)


def _shift1(a, k):
    return jnp.concatenate([jnp.zeros((a.shape[0], k), a.dtype), a[:, :-k]], axis=1)


def _router_body(x_ref, gw_ref, w_ref, dA_ref, dB_ref, tm_ref):
    x = x_ref[...]
    logits = lax.dot_general(x, gw_ref[...], (((1,), (1,)), ((), ())),
                             preferred_element_type=jnp.float32)
    mx = jnp.max(logits, axis=-1, keepdims=True)
    ex = jnp.exp(logits - mx)
    scores = ex / jnp.sum(ex, axis=-1, keepdims=True)
    iota = lax.broadcasted_iota(jnp.int32, scores.shape, 1)
    m1 = jnp.max(scores, axis=-1, keepdims=True)
    i1 = jnp.min(jnp.where(scores == m1, iota, E), axis=-1, keepdims=True)
    sel1 = iota == i1
    masked = jnp.where(sel1, -jnp.inf, scores)
    m2 = jnp.max(masked, axis=-1, keepdims=True)
    i2 = jnp.min(jnp.where(masked == m2, iota, E), axis=-1, keepdims=True)
    sel2 = iota == i2
    maskf = jnp.where(sel1 | sel2, 1.0, 0.0).astype(jnp.float32)

    acc = _shift0(maskf, 1)
    k = 1
    while k < N:
        acc = acc + _shift0(acc, k)
        k *= 2
    rank = acc
    counts = rank[N - 1:N, :] + maskf[N - 1:N, :]
    tiles = jnp.floor((counts + (GT - 1)) / GT)
    texc = _shift1(tiles, 1)
    texc = texc + _shift1(texc, 1)
    texc = texc + _shift1(texc, 2)
    texc = texc + _shift1(texc, 4)
    seg_off = GT * texc
    cum_incl = texc + tiles

    pos = seg_off + rank
    dA_ref[...] = jnp.sum(jnp.where(sel1, pos, 0.0), axis=1,
                          keepdims=True).astype(jnp.int32)
    dB_ref[...] = jnp.sum(jnp.where(sel2, pos, 0.0), axis=1,
                          keepdims=True).astype(jnp.int32)
    w_ref[...] = jnp.concatenate([m1, m2], axis=1)

    tio = lax.broadcasted_iota(jnp.int32, (1, 64), 1).astype(jnp.float32)
    tm = jnp.zeros((1, 64), jnp.float32)
    for e in range(E):
        tm = tm + jnp.where(tio >= cum_incl[:, e:e + 1], 1.0, 0.0)
    tm_ref[...] = jnp.minimum(tm, float(E - 1)).astype(jnp.int32)


def _gffn_body(tm_ref, xs_ref, gu_ref, dn_ref, ys_ref):
    xb = xs_ref[...]
    gu = lax.dot_general(xb, gu_ref[0], (((1,), (1,)), ((), ())),
                         preferred_element_type=jnp.float32)
    g = gu[:, :I]
    u = gu[:, I:]
    h = g * jax.nn.sigmoid(g) * u
    ys_ref[...] = lax.dot_general(h, dn_ref[0], (((1,), (1,)), ((), ())),
                                  preferred_element_type=jnp.float32)


def _shared_body(x_ref, sg_ref, su_ref, sd_ref, seg_ref, out_ref):
    xb = x_ref[...]
    g = lax.dot_general(xb, sg_ref[...], (((1,), (1,)), ((), ())),
                        preferred_element_type=jnp.float32)
    u = lax.dot_general(xb, su_ref[...], (((1,), (1,)), ((), ())),
                        preferred_element_type=jnp.float32)
    h = g * jax.nn.sigmoid(g) * u
    sh = lax.dot_general(h, sd_ref[...], (((1,), (1,)), ((), ())),
                         preferred_element_type=jnp.float32)
    gate_logit = jnp.sum(xb * seg_ref[...], axis=1, keepdims=True)
    out_ref[...] = jax.nn.sigmoid(gate_logit) * sh


def _combine_body(y0_ref, y1_ref, w_ref, sh_ref, out_ref):
    w = w_ref[...]
    out_ref[...] = (w[:, 0:1] * y0_ref[...] + w[:, 1:2] * y1_ref[...]
                    + sh_ref[...])


@functools.cache
def _sc_mesh():
    return plsc.VectorSubcoreMesh(core_axis_name="c", subcore_axis_name="s",
                                  num_cores=NC, num_subcores=NSUB)


def _dispatch_tokens(xf, dA, dB):
    @functools.partial(
        pl.kernel,
        out_type=jax.ShapeDtypeStruct((NS, H), jnp.float32),
        mesh=_sc_mesh(),
        scratch_types=[
            pltpu.VMEM((BPW,), jnp.int32),
            pltpu.VMEM((BPW,), jnp.int32),
            pltpu.VMEM((BPW, H), jnp.float32),
        ],
    )
    def scatter_kernel(x_hbm, dA_hbm, dB_hbm, xs_hbm, idxA_v, idxB_v, rows_v):
        wid = lax.axis_index("s") * NC + lax.axis_index("c")
        base = wid * BPW
        pltpu.sync_copy(dA_hbm.at[pl.ds(base, BPW)], idxA_v)
        pltpu.sync_copy(dB_hbm.at[pl.ds(base, BPW)], idxB_v)
        pltpu.sync_copy(x_hbm.at[pl.ds(base, BPW)], rows_v)
        pltpu.sync_copy(rows_v, xs_hbm.at[idxA_v])
        pltpu.sync_copy(rows_v, xs_hbm.at[idxB_v])

    return scatter_kernel(xf, dA, dB)


def _collect_rows(ys, dA, dB):
    @functools.partial(
        pl.kernel,
        out_type=(jax.ShapeDtypeStruct((N, H), jnp.float32),
                  jax.ShapeDtypeStruct((N, H), jnp.float32)),
        mesh=_sc_mesh(),
        scratch_types=[
            pltpu.VMEM((BPW,), jnp.int32),
            pltpu.VMEM((BPW, H), jnp.float32),
            pltpu.SemaphoreType.DMA,
        ],
    )
    def gather_kernel(ys_hbm, dA_hbm, dB_hbm, y0_hbm, y1_hbm, idx_v, rows_v, sem):
        wid = lax.axis_index("s") * NC + lax.axis_index("c")
        base = wid * BPW
        pltpu.sync_copy(dA_hbm.at[pl.ds(base, BPW)], idx_v)
        pltpu.async_copy(ys_hbm.at[idx_v], rows_v, sem).wait()
        pltpu.sync_copy(rows_v, y0_hbm.at[pl.ds(base, BPW)])
        pltpu.sync_copy(dB_hbm.at[pl.ds(base, BPW)], idx_v)
        pltpu.async_copy(ys_hbm.at[idx_v], rows_v, sem).wait()
        pltpu.sync_copy(rows_v, y1_hbm.at[pl.ds(base, BPW)])

    return gather_kernel(ys, dA, dB)


def kernel(x, gate_w, experts_gate_up, experts_down, shared_gate_w,
           shared_up_w, shared_down_w, shared_expert_gate_w):
    Bx, Tx, Hx = x.shape
    xf = x.reshape(Bx * Tx, Hx)

    w2, dA2, dB2, tm = pl.pallas_call(
        _router_body,
        grid=(1,),
        in_specs=[
            pl.BlockSpec((N, H), lambda i: (0, 0)),
            pl.BlockSpec((E, H), lambda i: (0, 0)),
        ],
        out_specs=[
            pl.BlockSpec((N, 2), lambda i: (0, 0)),
            pl.BlockSpec((N, 1), lambda i: (0, 0)),
            pl.BlockSpec((N, 1), lambda i: (0, 0)),
            pl.BlockSpec((1, 64), lambda i: (0, 0)),
        ],
        out_shape=[
            jax.ShapeDtypeStruct((N, 2), jnp.float32),
            jax.ShapeDtypeStruct((N, 1), jnp.int32),
            jax.ShapeDtypeStruct((N, 1), jnp.int32),
            jax.ShapeDtypeStruct((1, 64), jnp.int32),
        ],
    )(xf, gate_w)
    dA = dA2.reshape(N)
    dB = dB2.reshape(N)
    tile_map = tm.reshape(64)[:NTILES]

    sh = pl.pallas_call(
        _shared_body,
        grid=(N // BT,),
        in_specs=[
            pl.BlockSpec((BT, H), lambda t: (t, 0)),
            pl.BlockSpec((S, H), lambda t: (0, 0)),
            pl.BlockSpec((S, H), lambda t: (0, 0)),
            pl.BlockSpec((H, S), lambda t: (0, 0)),
            pl.BlockSpec((1, H), lambda t: (0, 0)),
        ],
        out_specs=pl.BlockSpec((BT, H), lambda t: (t, 0)),
        out_shape=jax.ShapeDtypeStruct((N, H), jnp.float32),
    )(xf, shared_gate_w, shared_up_w, shared_down_w, shared_expert_gate_w)

    xsf = _dispatch_tokens(xf, dA, dB)

    ys = pl.pallas_call(
        _gffn_body,
        grid_spec=pltpu.PrefetchScalarGridSpec(
            num_scalar_prefetch=1,
            grid=(NTILES,),
            in_specs=[
                pl.BlockSpec((GT, H), lambda t, tm_ref: (t, 0)),
                pl.BlockSpec((1, 2 * I, H), lambda t, tm_ref: (tm_ref[t], 0, 0)),
                pl.BlockSpec((1, H, I), lambda t, tm_ref: (tm_ref[t], 0, 0)),
            ],
            out_specs=pl.BlockSpec((GT, H), lambda t, tm_ref: (t, 0)),
        ),
        out_shape=jax.ShapeDtypeStruct((NS, H), jnp.float32),
    )(tile_map, xsf, experts_gate_up, experts_down)

    y0, y1 = _collect_rows(ys, dA, dB)

    out = pl.pallas_call(
        _combine_body,
        grid=(N // BT,),
        in_specs=[
            pl.BlockSpec((BT, H), lambda t: (t, 0)),
            pl.BlockSpec((BT, H), lambda t: (t, 0)),
            pl.BlockSpec((BT, 2), lambda t: (t, 0)),
            pl.BlockSpec((BT, H), lambda t: (t, 0)),
        ],
        out_specs=pl.BlockSpec((BT, H), lambda t: (t, 0)),
        out_shape=jax.ShapeDtypeStruct((N, H), jnp.float32),
    )(y0, y1, w2, sh)

    return out.reshape(Bx, Tx, Hx)


# speedup vs baseline: 1.9001x; 1.0522x over previous
import functools

import jax
import jax.numpy as jnp
from jax import lax
from jax.experimental import pallas as pl
from jax.experimental.pallas import tpu as pltpu
from jax.experimental.pallas import tpu_sc as plsc

H = 1024
E = 8
I = 1408
S = 2816
N = 2048
BT = 256
GT = 256
NTILES = N * 2 // GT + (E - 1)
NS = NTILES * GT
NC = 2
NSUB = 16
NW = NC * NSUB
BPW = N // NW


def _shift0(a, k):
    return jnp.concatenate([jnp.zeros((k, a.shape[1]), a.dtype), a[:-k]], axis=0)


def _shift1(a, k):
    return jnp.concatenate([jnp.zeros((a.shape[0], k), a.dtype), a[:, :-k]], axis=1)


def _router_body(x_ref, gw_ref, w_ref, dA_ref, dB_ref, tm_ref):
    x = x_ref[...]
    logits = lax.dot_general(x, gw_ref[...], (((1,), (1,)), ((), ())),
                             preferred_element_type=jnp.float32)
    mx = jnp.max(logits, axis=-1, keepdims=True)
    ex = jnp.exp(logits - mx)
    scores = ex / jnp.sum(ex, axis=-1, keepdims=True)
    iota = lax.broadcasted_iota(jnp.int32, scores.shape, 1)
    m1 = jnp.max(scores, axis=-1, keepdims=True)
    i1 = jnp.min(jnp.where(scores == m1, iota, E), axis=-1, keepdims=True)
    sel1 = iota == i1
    masked = jnp.where(sel1, -jnp.inf, scores)
    m2 = jnp.max(masked, axis=-1, keepdims=True)
    i2 = jnp.min(jnp.where(masked == m2, iota, E), axis=-1, keepdims=True)
    sel2 = iota == i2
    maskf = jnp.where(sel1 | sel2, 1.0, 0.0).astype(jnp.float32)

    acc = _shift0(maskf, 1)
    k = 1
    while k < N:
        acc = acc + _shift0(acc, k)
        k *= 2
    rank = acc
    counts = rank[N - 1:N, :] + maskf[N - 1:N, :]
    tiles = jnp.floor((counts + (GT - 1)) / GT)
    texc = _shift1(tiles, 1)
    texc = texc + _shift1(texc, 1)
    texc = texc + _shift1(texc, 2)
    texc = texc + _shift1(texc, 4)
    seg_off = GT * texc
    cum_incl = texc + tiles

    pos = seg_off + rank
    dA_ref[...] = jnp.sum(jnp.where(sel1, pos, 0.0), axis=1,
                          keepdims=True).astype(jnp.int32)
    dB_ref[...] = jnp.sum(jnp.where(sel2, pos, 0.0), axis=1,
                          keepdims=True).astype(jnp.int32)
    w_ref[...] = jnp.concatenate([m1, m2], axis=1)

    tio = lax.broadcasted_iota(jnp.int32, (1, 64), 1).astype(jnp.float32)
    tm = jnp.zeros((1, 64), jnp.float32)
    for e in range(E):
        tm = tm + jnp.where(tio >= cum_incl[:, e:e + 1], 1.0, 0.0)
    tm_ref[...] = jnp.minimum(tm, float(E - 1)).astype(jnp.int32)


def _gffn_body(tm_ref, xs_ref, gu_ref, dn_ref, ys_ref):
    xb = xs_ref[...]
    gu = lax.dot_general(xb, gu_ref[0], (((1,), (1,)), ((), ())),
                         preferred_element_type=jnp.float32)
    g = gu[:, :I]
    u = gu[:, I:]
    h = g * jax.nn.sigmoid(g) * u
    ys_ref[...] = lax.dot_general(h, dn_ref[0], (((1,), (1,)), ((), ())),
                                  preferred_element_type=jnp.float32)


def _shared_body(x_ref, sg_ref, su_ref, sd_ref, seg_ref, out_ref):
    s2 = pl.program_id(1)
    xb = x_ref[...]
    g = lax.dot_general(xb, sg_ref[...], (((1,), (1,)), ((), ())),
                        preferred_element_type=jnp.float32)
    u = lax.dot_general(xb, su_ref[...], (((1,), (1,)), ((), ())),
                        preferred_element_type=jnp.float32)
    h = g * jax.nn.sigmoid(g) * u
    sh = lax.dot_general(h, sd_ref[...], (((1,), (1,)), ((), ())),
                         preferred_element_type=jnp.float32)

    @pl.when(s2 == 0)
    def _():
        out_ref[...] = sh

    @pl.when(s2 == 1)
    def _():
        gate_logit = jnp.sum(xb * seg_ref[...], axis=1, keepdims=True)
        out_ref[...] = (out_ref[...] + sh) * jax.nn.sigmoid(gate_logit)


def _combine_body(y0_ref, y1_ref, w_ref, sh_ref, out_ref):
    w = w_ref[...]
    out_ref[...] = (w[:, 0:1] * y0_ref[...] + w[:, 1:2] * y1_ref[...]
                    + sh_ref[...])


@functools.cache
def _sc_mesh():
    return plsc.VectorSubcoreMesh(core_axis_name="c", subcore_axis_name="s",
                                  num_cores=NC, num_subcores=NSUB)


def _dispatch_tokens(xf, dA, dB):
    @functools.partial(
        pl.kernel,
        out_type=jax.ShapeDtypeStruct((NS, H), jnp.float32),
        mesh=_sc_mesh(),
        scratch_types=[
            pltpu.VMEM((BPW,), jnp.int32),
            pltpu.VMEM((BPW,), jnp.int32),
            pltpu.VMEM((BPW, H), jnp.float32),
        ],
    )
    def scatter_kernel(x_hbm, dA_hbm, dB_hbm, xs_hbm, idxA_v, idxB_v, rows_v):
        wid = lax.axis_index("s") * NC + lax.axis_index("c")
        base = wid * BPW
        pltpu.sync_copy(dA_hbm.at[pl.ds(base, BPW)], idxA_v)
        pltpu.sync_copy(dB_hbm.at[pl.ds(base, BPW)], idxB_v)
        pltpu.sync_copy(x_hbm.at[pl.ds(base, BPW)], rows_v)
        pltpu.sync_copy(rows_v, xs_hbm.at[idxA_v])
        pltpu.sync_copy(rows_v, xs_hbm.at[idxB_v])

    return scatter_kernel(xf, dA, dB)


def _collect_rows(ys, dA, dB):
    @functools.partial(
        pl.kernel,
        out_type=(jax.ShapeDtypeStruct((N, H), jnp.float32),
                  jax.ShapeDtypeStruct((N, H), jnp.float32)),
        mesh=_sc_mesh(),
        scratch_types=[
            pltpu.VMEM((BPW,), jnp.int32),
            pltpu.VMEM((BPW, H), jnp.float32),
            pltpu.SemaphoreType.DMA,
        ],
    )
    def gather_kernel(ys_hbm, dA_hbm, dB_hbm, y0_hbm, y1_hbm, idx_v, rows_v, sem):
        wid = lax.axis_index("s") * NC + lax.axis_index("c")
        base = wid * BPW
        pltpu.sync_copy(dA_hbm.at[pl.ds(base, BPW)], idx_v)
        pltpu.async_copy(ys_hbm.at[idx_v], rows_v, sem).wait()
        pltpu.sync_copy(rows_v, y0_hbm.at[pl.ds(base, BPW)])
        pltpu.sync_copy(dB_hbm.at[pl.ds(base, BPW)], idx_v)
        pltpu.async_copy(ys_hbm.at[idx_v], rows_v, sem).wait()
        pltpu.sync_copy(rows_v, y1_hbm.at[pl.ds(base, BPW)])

    return gather_kernel(ys, dA, dB)


def kernel(x, gate_w, experts_gate_up, experts_down, shared_gate_w,
           shared_up_w, shared_down_w, shared_expert_gate_w):
    Bx, Tx, Hx = x.shape
    xf = x.reshape(Bx * Tx, Hx)

    w2, dA2, dB2, tm = pl.pallas_call(
        _router_body,
        grid=(1,),
        in_specs=[
            pl.BlockSpec((N, H), lambda i: (0, 0)),
            pl.BlockSpec((E, H), lambda i: (0, 0)),
        ],
        out_specs=[
            pl.BlockSpec((N, 2), lambda i: (0, 0)),
            pl.BlockSpec((N, 1), lambda i: (0, 0)),
            pl.BlockSpec((N, 1), lambda i: (0, 0)),
            pl.BlockSpec((1, 64), lambda i: (0, 0)),
        ],
        out_shape=[
            jax.ShapeDtypeStruct((N, 2), jnp.float32),
            jax.ShapeDtypeStruct((N, 1), jnp.int32),
            jax.ShapeDtypeStruct((N, 1), jnp.int32),
            jax.ShapeDtypeStruct((1, 64), jnp.int32),
        ],
    )(xf, gate_w)
    dA = dA2.reshape(N)
    dB = dB2.reshape(N)
    tile_map = tm.reshape(64)[:NTILES]

    sh = pl.pallas_call(
        _shared_body,
        grid=(N // BT, 2),
        in_specs=[
            pl.BlockSpec((BT, H), lambda t, s2: (t, 0)),
            pl.BlockSpec((S // 2, H), lambda t, s2: (s2, 0)),
            pl.BlockSpec((S // 2, H), lambda t, s2: (s2, 0)),
            pl.BlockSpec((H, S // 2), lambda t, s2: (0, s2)),
            pl.BlockSpec((1, H), lambda t, s2: (0, 0)),
        ],
        out_specs=pl.BlockSpec((BT, H), lambda t, s2: (t, 0)),
        out_shape=jax.ShapeDtypeStruct((N, H), jnp.float32),
    )(xf, shared_gate_w, shared_up_w, shared_down_w, shared_expert_gate_w)

    xsf = _dispatch_tokens(xf, dA, dB)

    ys = pl.pallas_call(
        _gffn_body,
        grid_spec=pltpu.PrefetchScalarGridSpec(
            num_scalar_prefetch=1,
            grid=(NTILES,),
            in_specs=[
                pl.BlockSpec((GT, H), lambda t, tm_ref: (t, 0)),
                pl.BlockSpec((1, 2 * I, H), lambda t, tm_ref: (tm_ref[t], 0, 0)),
                pl.BlockSpec((1, H, I), lambda t, tm_ref: (tm_ref[t], 0, 0)),
            ],
            out_specs=pl.BlockSpec((GT, H), lambda t, tm_ref: (t, 0)),
        ),
        out_shape=jax.ShapeDtypeStruct((NS, H), jnp.float32),
    )(tile_map, xsf, experts_gate_up, experts_down)

    y0, y1 = _collect_rows(ys, dA, dB)

    out = pl.pallas_call(
        _combine_body,
        grid=(N // BT,),
        in_specs=[
            pl.BlockSpec((BT, H), lambda t: (t, 0)),
            pl.BlockSpec((BT, H), lambda t: (t, 0)),
            pl.BlockSpec((BT, 2), lambda t: (t, 0)),
            pl.BlockSpec((BT, H), lambda t: (t, 0)),
        ],
        out_specs=pl.BlockSpec((BT, H), lambda t: (t, 0)),
        out_shape=jax.ShapeDtypeStruct((N, H), jnp.float32),
    )(y0, y1, w2, sh)

    return out.reshape(Bx, Tx, Hx)


# speedup vs baseline: 1.9275x; 1.0144x over previous
import functools

import jax
import jax.numpy as jnp
from jax import lax
from jax.experimental import pallas as pl
from jax.experimental.pallas import tpu as pltpu
from jax.experimental.pallas import tpu_sc as plsc

H = 1024
E = 8
I = 1408
S = 2816
N = 2048
BT = 256
GT = 256
NTILES = N * 2 // GT + (E - 1)
NS = NTILES * GT
NC = 2
NSUB = 16
NW = NC * NSUB
BPW = N // NW


def _shift0(a, k):
    return jnp.concatenate([jnp.zeros((k, a.shape[1]), a.dtype), a[:-k]], axis=0)


def _shift1(a, k):
    return jnp.concatenate([jnp.zeros((a.shape[0], k), a.dtype), a[:, :-k]], axis=1)


def _router_body(x_ref, gw_ref, w_ref, dA_ref, dB_ref, tm_ref):
    x = x_ref[...]
    logits = lax.dot_general(x, gw_ref[...], (((1,), (1,)), ((), ())),
                             preferred_element_type=jnp.float32)
    mx = jnp.max(logits, axis=-1, keepdims=True)
    ex = jnp.exp(logits - mx)
    scores = ex / jnp.sum(ex, axis=-1, keepdims=True)
    iota = lax.broadcasted_iota(jnp.int32, scores.shape, 1)
    m1 = jnp.max(scores, axis=-1, keepdims=True)
    i1 = jnp.min(jnp.where(scores == m1, iota, E), axis=-1, keepdims=True)
    sel1 = iota == i1
    masked = jnp.where(sel1, -jnp.inf, scores)
    m2 = jnp.max(masked, axis=-1, keepdims=True)
    i2 = jnp.min(jnp.where(masked == m2, iota, E), axis=-1, keepdims=True)
    sel2 = iota == i2
    maskf = jnp.where(sel1 | sel2, 1.0, 0.0).astype(jnp.float32)

    acc = _shift0(maskf, 1)
    k = 1
    while k < N:
        acc = acc + _shift0(acc, k)
        k *= 2
    rank = acc
    counts = rank[N - 1:N, :] + maskf[N - 1:N, :]
    tiles = jnp.floor((counts + (GT - 1)) / GT)
    texc = _shift1(tiles, 1)
    texc = texc + _shift1(texc, 1)
    texc = texc + _shift1(texc, 2)
    texc = texc + _shift1(texc, 4)
    seg_off = GT * texc
    cum_incl = texc + tiles

    pos = seg_off + rank
    dA_ref[...] = jnp.sum(jnp.where(sel1, pos, 0.0), axis=1,
                          keepdims=True).astype(jnp.int32)
    dB_ref[...] = jnp.sum(jnp.where(sel2, pos, 0.0), axis=1,
                          keepdims=True).astype(jnp.int32)
    w_ref[...] = jnp.concatenate([m1, m2], axis=1)

    tio = lax.broadcasted_iota(jnp.int32, (1, 64), 1).astype(jnp.float32)
    tm = jnp.zeros((1, 64), jnp.float32)
    for e in range(E):
        tm = tm + jnp.where(tio >= cum_incl[:, e:e + 1], 1.0, 0.0)
    tm_ref[...] = jnp.minimum(tm, float(E - 1)).astype(jnp.int32)


def _gffn_body(tm_ref, xs_ref, gu_ref, dn_ref, ys_ref, acc_ref):
    kc = pl.program_id(1)
    xb = xs_ref[...]
    part = lax.dot_general(xb, gu_ref[0], (((1,), (1,)), ((), ())),
                           preferred_element_type=jnp.float32)

    @pl.when(kc == 0)
    def _():
        acc_ref[...] = part

    @pl.when(kc == 1)
    def _():
        gu = acc_ref[...] + part
        g = gu[:, :I]
        u = gu[:, I:]
        h = g * jax.nn.sigmoid(g) * u
        ys_ref[...] = lax.dot_general(h, dn_ref[0], (((1,), (1,)), ((), ())),
                                      preferred_element_type=jnp.float32)


def _shared_body(x_ref, sg_ref, su_ref, sd_ref, seg_ref, out_ref):
    s2 = pl.program_id(0)
    t = pl.program_id(1)
    tb = pl.ds(t * BT, BT)
    xb = x_ref[...]
    g = lax.dot_general(xb, sg_ref[...], (((1,), (1,)), ((), ())),
                        preferred_element_type=jnp.float32)
    u = lax.dot_general(xb, su_ref[...], (((1,), (1,)), ((), ())),
                        preferred_element_type=jnp.float32)
    h = g * jax.nn.sigmoid(g) * u
    sh = lax.dot_general(h, sd_ref[...], (((1,), (1,)), ((), ())),
                         preferred_element_type=jnp.float32)

    @pl.when(s2 == 0)
    def _():
        out_ref[tb, :] = sh

    @pl.when(s2 == 1)
    def _():
        gate_logit = jnp.sum(xb * seg_ref[...], axis=1, keepdims=True)
        out_ref[tb, :] = (out_ref[tb, :] + sh) * jax.nn.sigmoid(gate_logit)


def _combine_body(y0_ref, y1_ref, w_ref, sh_ref, out_ref):
    w = w_ref[...]
    out_ref[...] = (w[:, 0:1] * y0_ref[...] + w[:, 1:2] * y1_ref[...]
                    + sh_ref[...])


@functools.cache
def _sc_mesh():
    return plsc.VectorSubcoreMesh(core_axis_name="c", subcore_axis_name="s",
                                  num_cores=NC, num_subcores=NSUB)


def _dispatch_tokens(xf, dA, dB):
    @functools.partial(
        pl.kernel,
        out_type=jax.ShapeDtypeStruct((NS, H), jnp.float32),
        mesh=_sc_mesh(),
        scratch_types=[
            pltpu.VMEM((BPW,), jnp.int32),
            pltpu.VMEM((BPW,), jnp.int32),
            pltpu.VMEM((BPW, H), jnp.float32),
        ],
    )
    def scatter_kernel(x_hbm, dA_hbm, dB_hbm, xs_hbm, idxA_v, idxB_v, rows_v):
        wid = lax.axis_index("s") * NC + lax.axis_index("c")
        base = wid * BPW
        pltpu.sync_copy(dA_hbm.at[pl.ds(base, BPW)], idxA_v)
        pltpu.sync_copy(dB_hbm.at[pl.ds(base, BPW)], idxB_v)
        pltpu.sync_copy(x_hbm.at[pl.ds(base, BPW)], rows_v)
        pltpu.sync_copy(rows_v, xs_hbm.at[idxA_v])
        pltpu.sync_copy(rows_v, xs_hbm.at[idxB_v])

    return scatter_kernel(xf, dA, dB)


def _collect_rows(ys, dA, dB):
    @functools.partial(
        pl.kernel,
        out_type=(jax.ShapeDtypeStruct((N, H), jnp.float32),
                  jax.ShapeDtypeStruct((N, H), jnp.float32)),
        mesh=_sc_mesh(),
        scratch_types=[
            pltpu.VMEM((BPW,), jnp.int32),
            pltpu.VMEM((BPW, H), jnp.float32),
            pltpu.SemaphoreType.DMA,
        ],
    )
    def gather_kernel(ys_hbm, dA_hbm, dB_hbm, y0_hbm, y1_hbm, idx_v, rows_v, sem):
        wid = lax.axis_index("s") * NC + lax.axis_index("c")
        base = wid * BPW
        pltpu.sync_copy(dA_hbm.at[pl.ds(base, BPW)], idx_v)
        pltpu.async_copy(ys_hbm.at[idx_v], rows_v, sem).wait()
        pltpu.sync_copy(rows_v, y0_hbm.at[pl.ds(base, BPW)])
        pltpu.sync_copy(dB_hbm.at[pl.ds(base, BPW)], idx_v)
        pltpu.async_copy(ys_hbm.at[idx_v], rows_v, sem).wait()
        pltpu.sync_copy(rows_v, y1_hbm.at[pl.ds(base, BPW)])

    return gather_kernel(ys, dA, dB)


def kernel(x, gate_w, experts_gate_up, experts_down, shared_gate_w,
           shared_up_w, shared_down_w, shared_expert_gate_w):
    Bx, Tx, Hx = x.shape
    xf = x.reshape(Bx * Tx, Hx)

    w2, dA2, dB2, tm = pl.pallas_call(
        _router_body,
        grid=(1,),
        in_specs=[
            pl.BlockSpec((N, H), lambda i: (0, 0)),
            pl.BlockSpec((E, H), lambda i: (0, 0)),
        ],
        out_specs=[
            pl.BlockSpec((N, 2), lambda i: (0, 0)),
            pl.BlockSpec((N, 1), lambda i: (0, 0)),
            pl.BlockSpec((N, 1), lambda i: (0, 0)),
            pl.BlockSpec((1, 64), lambda i: (0, 0)),
        ],
        out_shape=[
            jax.ShapeDtypeStruct((N, 2), jnp.float32),
            jax.ShapeDtypeStruct((N, 1), jnp.int32),
            jax.ShapeDtypeStruct((N, 1), jnp.int32),
            jax.ShapeDtypeStruct((1, 64), jnp.int32),
        ],
    )(xf, gate_w)
    dA = dA2.reshape(N)
    dB = dB2.reshape(N)
    tile_map = tm.reshape(64)[:NTILES]

    sh = pl.pallas_call(
        _shared_body,
        grid=(2, N // BT),
        in_specs=[
            pl.BlockSpec((BT, H), lambda s2, t: (t, 0)),
            pl.BlockSpec((S // 2, H), lambda s2, t: (s2, 0)),
            pl.BlockSpec((S // 2, H), lambda s2, t: (s2, 0)),
            pl.BlockSpec((H, S // 2), lambda s2, t: (0, s2)),
            pl.BlockSpec((1, H), lambda s2, t: (0, 0)),
        ],
        out_specs=pl.BlockSpec((N, H), lambda s2, t: (0, 0)),
        out_shape=jax.ShapeDtypeStruct((N, H), jnp.float32),
    )(xf, shared_gate_w, shared_up_w, shared_down_w, shared_expert_gate_w)

    xsf = _dispatch_tokens(xf, dA, dB)

    ys = pl.pallas_call(
        _gffn_body,
        grid_spec=pltpu.PrefetchScalarGridSpec(
            num_scalar_prefetch=1,
            grid=(NTILES, 2),
            in_specs=[
                pl.BlockSpec((GT, H // 2), lambda t, kc, tm_ref: (t, kc)),
                pl.BlockSpec((1, 2 * I, H // 2),
                             lambda t, kc, tm_ref: (tm_ref[t], 0, kc)),
                pl.BlockSpec((1, H, I), lambda t, kc, tm_ref: (tm_ref[t], 0, 0)),
            ],
            out_specs=pl.BlockSpec((GT, H), lambda t, kc, tm_ref: (t, 0)),
            scratch_shapes=[pltpu.VMEM((GT, 2 * I), jnp.float32)],
        ),
        out_shape=jax.ShapeDtypeStruct((NS, H), jnp.float32),
    )(tile_map, xsf, experts_gate_up, experts_down)

    y0, y1 = _collect_rows(ys, dA, dB)

    out = pl.pallas_call(
        _combine_body,
        grid=(N // BT,),
        in_specs=[
            pl.BlockSpec((BT, H), lambda t: (t, 0)),
            pl.BlockSpec((BT, H), lambda t: (t, 0)),
            pl.BlockSpec((BT, 2), lambda t: (t, 0)),
            pl.BlockSpec((BT, H), lambda t: (t, 0)),
        ],
        out_specs=pl.BlockSpec((BT, H), lambda t: (t, 0)),
        out_shape=jax.ShapeDtypeStruct((N, H), jnp.float32),
    )(y0, y1, w2, sh)

    return out.reshape(Bx, Tx, Hx)


# speedup vs baseline: 1.9585x; 1.0161x over previous
import functools

import jax
import jax.numpy as jnp
from jax import lax
from jax.experimental import pallas as pl
from jax.experimental.pallas import tpu as pltpu
from jax.experimental.pallas import tpu_sc as plsc

H = 1024
E = 8
I = 1408
S = 2816
N = 2048
BT = 256
GT = 256
NTILES = N * 2 // GT + (E - 1)
NS = NTILES * GT
NC = 2
NSUB = 16
NW = NC * NSUB
BPW = N // NW


def _shift0(a, k):
    return jnp.concatenate([jnp.zeros((k, a.shape[1]), a.dtype), a[:-k]], axis=0)


def _shift1(a, k):
    return jnp.concatenate([jnp.zeros((a.shape[0], k), a.dtype), a[:, :-k]], axis=1)


def _router_body(x_ref, gw_ref, w_ref, dA_ref, dB_ref, tm_ref):
    x = x_ref[...]
    logits = lax.dot_general(x, gw_ref[...], (((1,), (1,)), ((), ())),
                             preferred_element_type=jnp.float32)
    mx = jnp.max(logits, axis=-1, keepdims=True)
    ex = jnp.exp(logits - mx)
    scores = ex / jnp.sum(ex, axis=-1, keepdims=True)
    iota = lax.broadcasted_iota(jnp.int32, scores.shape, 1)
    m1 = jnp.max(scores, axis=-1, keepdims=True)
    i1 = jnp.min(jnp.where(scores == m1, iota, E), axis=-1, keepdims=True)
    sel1 = iota == i1
    masked = jnp.where(sel1, -jnp.inf, scores)
    m2 = jnp.max(masked, axis=-1, keepdims=True)
    i2 = jnp.min(jnp.where(masked == m2, iota, E), axis=-1, keepdims=True)
    sel2 = iota == i2
    maskf = jnp.where(sel1 | sel2, 1.0, 0.0).astype(jnp.float32)

    acc = _shift0(maskf, 1)
    k = 1
    while k < N:
        acc = acc + _shift0(acc, k)
        k *= 2
    rank = acc
    counts = rank[N - 1:N, :] + maskf[N - 1:N, :]
    tiles = jnp.floor((counts + (GT - 1)) / GT)
    texc = _shift1(tiles, 1)
    texc = texc + _shift1(texc, 1)
    texc = texc + _shift1(texc, 2)
    texc = texc + _shift1(texc, 4)
    seg_off = GT * texc
    cum_incl = texc + tiles

    pos = seg_off + rank
    dA_ref[...] = jnp.sum(jnp.where(sel1, pos, 0.0), axis=1,
                          keepdims=True).astype(jnp.int32)
    dB_ref[...] = jnp.sum(jnp.where(sel2, pos, 0.0), axis=1,
                          keepdims=True).astype(jnp.int32)
    w_ref[...] = jnp.concatenate([m1, m2], axis=1)

    tio = lax.broadcasted_iota(jnp.int32, (1, 64), 1).astype(jnp.float32)
    tm = jnp.zeros((1, 64), jnp.float32)
    for e in range(E):
        tm = tm + jnp.where(tio >= cum_incl[:, e:e + 1], 1.0, 0.0)
    tm_ref[...] = jnp.minimum(tm, float(E - 1)).astype(jnp.int32)


def _gffn_body(tm_ref, xs_ref, gu_ref, dn_ref, ys_ref, acc_ref):
    kc = pl.program_id(1)
    xb = xs_ref[...]
    part = lax.dot_general(xb, gu_ref[0], (((1,), (1,)), ((), ())),
                           preferred_element_type=jnp.float32)

    @pl.when(kc == 0)
    def _():
        acc_ref[...] = part * jax.nn.sigmoid(part)

    @pl.when(kc == 1)
    def _():
        h = acc_ref[...] * part
        ys_ref[...] = lax.dot_general(h, dn_ref[0], (((1,), (1,)), ((), ())),
                                      preferred_element_type=jnp.float32)


def _shared_body(x_ref, sg_ref, su_ref, sd_ref, seg_ref, out_ref):
    s2 = pl.program_id(0)
    t = pl.program_id(1)
    tb = pl.ds(t * BT, BT)
    xb = x_ref[...]
    g = lax.dot_general(xb, sg_ref[...], (((1,), (1,)), ((), ())),
                        preferred_element_type=jnp.float32)
    u = lax.dot_general(xb, su_ref[...], (((1,), (1,)), ((), ())),
                        preferred_element_type=jnp.float32)
    h = g * jax.nn.sigmoid(g) * u
    sh = lax.dot_general(h, sd_ref[...], (((1,), (1,)), ((), ())),
                         preferred_element_type=jnp.float32)

    @pl.when(s2 == 0)
    def _():
        out_ref[tb, :] = sh

    @pl.when(s2 == 1)
    def _():
        gate_logit = jnp.sum(xb * seg_ref[...], axis=1, keepdims=True)
        out_ref[tb, :] = (out_ref[tb, :] + sh) * jax.nn.sigmoid(gate_logit)


def _combine_body(y0_ref, y1_ref, w_ref, sh_ref, out_ref):
    w = w_ref[...]
    out_ref[...] = (w[:, 0:1] * y0_ref[...] + w[:, 1:2] * y1_ref[...]
                    + sh_ref[...])


@functools.cache
def _sc_mesh():
    return plsc.VectorSubcoreMesh(core_axis_name="c", subcore_axis_name="s",
                                  num_cores=NC, num_subcores=NSUB)


def _dispatch_tokens(xf, dA, dB):
    @functools.partial(
        pl.kernel,
        out_type=jax.ShapeDtypeStruct((NS, H), jnp.float32),
        mesh=_sc_mesh(),
        scratch_types=[
            pltpu.VMEM((BPW,), jnp.int32),
            pltpu.VMEM((BPW,), jnp.int32),
            pltpu.VMEM((BPW, H), jnp.float32),
        ],
    )
    def scatter_kernel(x_hbm, dA_hbm, dB_hbm, xs_hbm, idxA_v, idxB_v, rows_v):
        wid = lax.axis_index("s") * NC + lax.axis_index("c")
        base = wid * BPW
        pltpu.sync_copy(dA_hbm.at[pl.ds(base, BPW)], idxA_v)
        pltpu.sync_copy(dB_hbm.at[pl.ds(base, BPW)], idxB_v)
        pltpu.sync_copy(x_hbm.at[pl.ds(base, BPW)], rows_v)
        pltpu.sync_copy(rows_v, xs_hbm.at[idxA_v])
        pltpu.sync_copy(rows_v, xs_hbm.at[idxB_v])

    return scatter_kernel(xf, dA, dB)


def _collect_rows(ys, dA, dB):
    @functools.partial(
        pl.kernel,
        out_type=(jax.ShapeDtypeStruct((N, H), jnp.float32),
                  jax.ShapeDtypeStruct((N, H), jnp.float32)),
        mesh=_sc_mesh(),
        scratch_types=[
            pltpu.VMEM((BPW,), jnp.int32),
            pltpu.VMEM((BPW, H), jnp.float32),
            pltpu.SemaphoreType.DMA,
        ],
    )
    def gather_kernel(ys_hbm, dA_hbm, dB_hbm, y0_hbm, y1_hbm, idx_v, rows_v, sem):
        wid = lax.axis_index("s") * NC + lax.axis_index("c")
        base = wid * BPW
        pltpu.sync_copy(dA_hbm.at[pl.ds(base, BPW)], idx_v)
        pltpu.async_copy(ys_hbm.at[idx_v], rows_v, sem).wait()
        pltpu.sync_copy(rows_v, y0_hbm.at[pl.ds(base, BPW)])
        pltpu.sync_copy(dB_hbm.at[pl.ds(base, BPW)], idx_v)
        pltpu.async_copy(ys_hbm.at[idx_v], rows_v, sem).wait()
        pltpu.sync_copy(rows_v, y1_hbm.at[pl.ds(base, BPW)])

    return gather_kernel(ys, dA, dB)


def kernel(x, gate_w, experts_gate_up, experts_down, shared_gate_w,
           shared_up_w, shared_down_w, shared_expert_gate_w):
    Bx, Tx, Hx = x.shape
    xf = x.reshape(Bx * Tx, Hx)

    w2, dA2, dB2, tm = pl.pallas_call(
        _router_body,
        grid=(1,),
        in_specs=[
            pl.BlockSpec((N, H), lambda i: (0, 0)),
            pl.BlockSpec((E, H), lambda i: (0, 0)),
        ],
        out_specs=[
            pl.BlockSpec((N, 2), lambda i: (0, 0)),
            pl.BlockSpec((N, 1), lambda i: (0, 0)),
            pl.BlockSpec((N, 1), lambda i: (0, 0)),
            pl.BlockSpec((1, 64), lambda i: (0, 0)),
        ],
        out_shape=[
            jax.ShapeDtypeStruct((N, 2), jnp.float32),
            jax.ShapeDtypeStruct((N, 1), jnp.int32),
            jax.ShapeDtypeStruct((N, 1), jnp.int32),
            jax.ShapeDtypeStruct((1, 64), jnp.int32),
        ],
    )(xf, gate_w)
    dA = dA2.reshape(N)
    dB = dB2.reshape(N)
    tile_map = tm.reshape(64)[:NTILES]

    sh = pl.pallas_call(
        _shared_body,
        grid=(2, N // BT),
        in_specs=[
            pl.BlockSpec((BT, H), lambda s2, t: (t, 0)),
            pl.BlockSpec((S // 2, H), lambda s2, t: (s2, 0)),
            pl.BlockSpec((S // 2, H), lambda s2, t: (s2, 0)),
            pl.BlockSpec((H, S // 2), lambda s2, t: (0, s2)),
            pl.BlockSpec((1, H), lambda s2, t: (0, 0)),
        ],
        out_specs=pl.BlockSpec((N, H), lambda s2, t: (0, 0)),
        out_shape=jax.ShapeDtypeStruct((N, H), jnp.float32),
    )(xf, shared_gate_w, shared_up_w, shared_down_w, shared_expert_gate_w)

    xsf = _dispatch_tokens(xf, dA, dB)

    ys = pl.pallas_call(
        _gffn_body,
        grid_spec=pltpu.PrefetchScalarGridSpec(
            num_scalar_prefetch=1,
            grid=(NTILES, 2),
            in_specs=[
                pl.BlockSpec((GT, H), lambda t, kc, tm_ref: (t, 0)),
                pl.BlockSpec((1, I, H), lambda t, kc, tm_ref: (tm_ref[t], kc, 0)),
                pl.BlockSpec((1, H, I), lambda t, kc, tm_ref: (tm_ref[t], 0, 0)),
            ],
            out_specs=pl.BlockSpec((GT, H), lambda t, kc, tm_ref: (t, 0)),
            scratch_shapes=[pltpu.VMEM((GT, I), jnp.float32)],
        ),
        out_shape=jax.ShapeDtypeStruct((NS, H), jnp.float32),
    )(tile_map, xsf, experts_gate_up, experts_down)

    y0, y1 = _collect_rows(ys, dA, dB)

    out = pl.pallas_call(
        _combine_body,
        grid=(N // BT,),
        in_specs=[
            pl.BlockSpec((BT, H), lambda t: (t, 0)),
            pl.BlockSpec((BT, H), lambda t: (t, 0)),
            pl.BlockSpec((BT, 2), lambda t: (t, 0)),
            pl.BlockSpec((BT, H), lambda t: (t, 0)),
        ],
        out_specs=pl.BlockSpec((BT, H), lambda t: (t, 0)),
        out_shape=jax.ShapeDtypeStruct((N, H), jnp.float32),
    )(y0, y1, w2, sh)

    return out.reshape(Bx, Tx, Hx)


# speedup vs baseline: 2.4509x; 1.2514x over previous
import functools

import jax
import jax.numpy as jnp
from jax import lax
from jax.experimental import pallas as pl
from jax.experimental.pallas import tpu as pltpu
from jax.experimental.pallas import tpu_sc as plsc

H = 1024
E = 8
I = 1408
S = 2816
N = 2048
BT = 256
GT = 512
NTILES = N * 2 // GT + (E - 1)
NS = NTILES * GT
NC = 2
NSUB = 16
NW = NC * NSUB
BPW = N // NW


def _shift0(a, k):
    return jnp.concatenate([jnp.zeros((k, a.shape[1]), a.dtype), a[:-k]], axis=0)


def _shift1(a, k):
    return jnp.concatenate([jnp.zeros((a.shape[0], k), a.dtype), a[:, :-k]], axis=1)


def _router_body(x_ref, gw_ref, w_ref, dA_ref, dB_ref, tm_ref):
    x = x_ref[...]
    logits = lax.dot_general(x, gw_ref[...], (((1,), (1,)), ((), ())),
                             preferred_element_type=jnp.float32)
    mx = jnp.max(logits, axis=-1, keepdims=True)
    ex = jnp.exp(logits - mx)
    scores = ex / jnp.sum(ex, axis=-1, keepdims=True)
    iota = lax.broadcasted_iota(jnp.int32, scores.shape, 1)
    m1 = jnp.max(scores, axis=-1, keepdims=True)
    i1 = jnp.min(jnp.where(scores == m1, iota, E), axis=-1, keepdims=True)
    sel1 = iota == i1
    masked = jnp.where(sel1, -jnp.inf, scores)
    m2 = jnp.max(masked, axis=-1, keepdims=True)
    i2 = jnp.min(jnp.where(masked == m2, iota, E), axis=-1, keepdims=True)
    sel2 = iota == i2
    maskf = jnp.where(sel1 | sel2, 1.0, 0.0).astype(jnp.float32)

    acc = _shift0(maskf, 1)
    k = 1
    while k < N:
        acc = acc + _shift0(acc, k)
        k *= 2
    rank = acc
    counts = rank[N - 1:N, :] + maskf[N - 1:N, :]
    tiles = jnp.floor((counts + (GT - 1)) / GT)
    texc = _shift1(tiles, 1)
    texc = texc + _shift1(texc, 1)
    texc = texc + _shift1(texc, 2)
    texc = texc + _shift1(texc, 4)
    seg_off = GT * texc
    cum_incl = texc + tiles

    pos = seg_off + rank
    dA_ref[...] = jnp.sum(jnp.where(sel1, pos, 0.0), axis=1,
                          keepdims=True).astype(jnp.int32)
    dB_ref[...] = jnp.sum(jnp.where(sel2, pos, 0.0), axis=1,
                          keepdims=True).astype(jnp.int32)
    w_ref[...] = jnp.concatenate([m1, m2], axis=1)

    tio = lax.broadcasted_iota(jnp.int32, (1, 64), 1).astype(jnp.float32)
    tm = jnp.zeros((1, 64), jnp.float32)
    for e in range(E):
        tm = tm + jnp.where(tio >= cum_incl[:, e:e + 1], 1.0, 0.0)
    tm_ref[...] = tm.astype(jnp.int32)


def _gffn_body(tm_ref, xs_ref, gu_ref, dn_ref, ys_ref):
    t = pl.program_id(0)

    @pl.when(tm_ref[t] < E)
    def _():
        xb = xs_ref[...]
        gu = lax.dot_general(xb, gu_ref[0], (((1,), (1,)), ((), ())),
                             preferred_element_type=jnp.float32)
        g = gu[:, :I]
        u = gu[:, I:]
        h = g * jax.nn.sigmoid(g) * u
        ys_ref[...] = lax.dot_general(h, dn_ref[0], (((1,), (1,)), ((), ())),
                                      preferred_element_type=jnp.float32)


def _shared_body(x_ref, sg_ref, su_ref, sd_ref, seg_ref, out_ref):
    s2 = pl.program_id(0)
    t = pl.program_id(1)
    tb = pl.ds(t * BT, BT)
    xb = x_ref[...]
    g = lax.dot_general(xb, sg_ref[...], (((1,), (1,)), ((), ())),
                        preferred_element_type=jnp.float32)
    u = lax.dot_general(xb, su_ref[...], (((1,), (1,)), ((), ())),
                        preferred_element_type=jnp.float32)
    h = g * jax.nn.sigmoid(g) * u
    sh = lax.dot_general(h, sd_ref[...], (((1,), (1,)), ((), ())),
                         preferred_element_type=jnp.float32)

    @pl.when(s2 == 0)
    def _():
        out_ref[tb, :] = sh

    @pl.when(s2 == 1)
    def _():
        gate_logit = jnp.sum(xb * seg_ref[...], axis=1, keepdims=True)
        out_ref[tb, :] = (out_ref[tb, :] + sh) * jax.nn.sigmoid(gate_logit)


def _combine_body(y0_ref, y1_ref, w_ref, sh_ref, out_ref):
    w = w_ref[...]
    out_ref[...] = (w[:, 0:1] * y0_ref[...] + w[:, 1:2] * y1_ref[...]
                    + sh_ref[...])


@functools.cache
def _sc_mesh():
    return plsc.VectorSubcoreMesh(core_axis_name="c", subcore_axis_name="s",
                                  num_cores=NC, num_subcores=NSUB)


def _dispatch_tokens(xf, dA, dB):
    @functools.partial(
        pl.kernel,
        out_type=jax.ShapeDtypeStruct((NS, H), jnp.float32),
        mesh=_sc_mesh(),
        scratch_types=[
            pltpu.VMEM((BPW,), jnp.int32),
            pltpu.VMEM((BPW,), jnp.int32),
            pltpu.VMEM((BPW, H), jnp.float32),
        ],
    )
    def scatter_kernel(x_hbm, dA_hbm, dB_hbm, xs_hbm, idxA_v, idxB_v, rows_v):
        wid = lax.axis_index("s") * NC + lax.axis_index("c")
        base = wid * BPW
        pltpu.sync_copy(dA_hbm.at[pl.ds(base, BPW)], idxA_v)
        pltpu.sync_copy(dB_hbm.at[pl.ds(base, BPW)], idxB_v)
        pltpu.sync_copy(x_hbm.at[pl.ds(base, BPW)], rows_v)
        pltpu.sync_copy(rows_v, xs_hbm.at[idxA_v])
        pltpu.sync_copy(rows_v, xs_hbm.at[idxB_v])

    return scatter_kernel(xf, dA, dB)


def _collect_rows(ys, dA, dB):
    @functools.partial(
        pl.kernel,
        out_type=(jax.ShapeDtypeStruct((N, H), jnp.float32),
                  jax.ShapeDtypeStruct((N, H), jnp.float32)),
        mesh=_sc_mesh(),
        scratch_types=[
            pltpu.VMEM((BPW,), jnp.int32),
            pltpu.VMEM((BPW, H), jnp.float32),
            pltpu.SemaphoreType.DMA,
        ],
    )
    def gather_kernel(ys_hbm, dA_hbm, dB_hbm, y0_hbm, y1_hbm, idx_v, rows_v, sem):
        wid = lax.axis_index("s") * NC + lax.axis_index("c")
        base = wid * BPW
        pltpu.sync_copy(dA_hbm.at[pl.ds(base, BPW)], idx_v)
        pltpu.async_copy(ys_hbm.at[idx_v], rows_v, sem).wait()
        pltpu.sync_copy(rows_v, y0_hbm.at[pl.ds(base, BPW)])
        pltpu.sync_copy(dB_hbm.at[pl.ds(base, BPW)], idx_v)
        pltpu.async_copy(ys_hbm.at[idx_v], rows_v, sem).wait()
        pltpu.sync_copy(rows_v, y1_hbm.at[pl.ds(base, BPW)])

    return gather_kernel(ys, dA, dB)


def kernel(x, gate_w, experts_gate_up, experts_down, shared_gate_w,
           shared_up_w, shared_down_w, shared_expert_gate_w):
    Bx, Tx, Hx = x.shape
    xf = x.reshape(Bx * Tx, Hx)

    w2, dA2, dB2, tm = pl.pallas_call(
        _router_body,
        grid=(1,),
        in_specs=[
            pl.BlockSpec((N, H), lambda i: (0, 0)),
            pl.BlockSpec((E, H), lambda i: (0, 0)),
        ],
        out_specs=[
            pl.BlockSpec((N, 2), lambda i: (0, 0)),
            pl.BlockSpec((N, 1), lambda i: (0, 0)),
            pl.BlockSpec((N, 1), lambda i: (0, 0)),
            pl.BlockSpec((1, 64), lambda i: (0, 0)),
        ],
        out_shape=[
            jax.ShapeDtypeStruct((N, 2), jnp.float32),
            jax.ShapeDtypeStruct((N, 1), jnp.int32),
            jax.ShapeDtypeStruct((N, 1), jnp.int32),
            jax.ShapeDtypeStruct((1, 64), jnp.int32),
        ],
    )(xf, gate_w)
    dA = dA2.reshape(N)
    dB = dB2.reshape(N)
    tile_map = tm.reshape(64)[:NTILES]

    sh = pl.pallas_call(
        _shared_body,
        grid=(2, N // BT),
        in_specs=[
            pl.BlockSpec((BT, H), lambda s2, t: (t, 0)),
            pl.BlockSpec((S // 2, H), lambda s2, t: (s2, 0)),
            pl.BlockSpec((S // 2, H), lambda s2, t: (s2, 0)),
            pl.BlockSpec((H, S // 2), lambda s2, t: (0, s2)),
            pl.BlockSpec((1, H), lambda s2, t: (0, 0)),
        ],
        out_specs=pl.BlockSpec((N, H), lambda s2, t: (0, 0)),
        out_shape=jax.ShapeDtypeStruct((N, H), jnp.float32),
    )(xf, shared_gate_w, shared_up_w, shared_down_w, shared_expert_gate_w)

    xsf = _dispatch_tokens(xf, dA, dB)

    ys = pl.pallas_call(
        _gffn_body,
        grid_spec=pltpu.PrefetchScalarGridSpec(
            num_scalar_prefetch=1,
            grid=(NTILES,),
            in_specs=[
                pl.BlockSpec((GT, H), lambda t, tm_ref: (t, 0)),
                pl.BlockSpec((1, 2 * I, H),
                             lambda t, tm_ref: (jnp.minimum(tm_ref[t], E - 1), 0, 0)),
                pl.BlockSpec((1, H, I),
                             lambda t, tm_ref: (jnp.minimum(tm_ref[t], E - 1), 0, 0)),
            ],
            out_specs=pl.BlockSpec((GT, H), lambda t, tm_ref: (t, 0)),
        ),
        out_shape=jax.ShapeDtypeStruct((NS, H), jnp.float32),
    )(tile_map, xsf, experts_gate_up, experts_down)

    y0, y1 = _collect_rows(ys, dA, dB)

    out = pl.pallas_call(
        _combine_body,
        grid=(N // BT,),
        in_specs=[
            pl.BlockSpec((BT, H), lambda t: (t, 0)),
            pl.BlockSpec((BT, H), lambda t: (t, 0)),
            pl.BlockSpec((BT, 2), lambda t: (t, 0)),
            pl.BlockSpec((BT, H), lambda t: (t, 0)),
        ],
        out_specs=pl.BlockSpec((BT, H), lambda t: (t, 0)),
        out_shape=jax.ShapeDtypeStruct((N, H), jnp.float32),
    )(y0, y1, w2, sh)

    return out.reshape(Bx, Tx, Hx)


# speedup vs baseline: 2.5130x; 1.0253x over previous
import functools

import jax
import jax.numpy as jnp
from jax import lax
from jax.experimental import pallas as pl
from jax.experimental.pallas import tpu as pltpu
from jax.experimental.pallas import tpu_sc as plsc

H = 1024
E = 8
I = 1408
S = 2816
N = 2048
BT = 512
GT = 512
NTILES = N * 2 // GT + (E - 1)
NS = NTILES * GT
NC = 2
NSUB = 16
NW = NC * NSUB
BPW = N // NW


def _shift0(a, k):
    return jnp.concatenate([jnp.zeros((k, a.shape[1]), a.dtype), a[:-k]], axis=0)


def _shift1(a, k):
    return jnp.concatenate([jnp.zeros((a.shape[0], k), a.dtype), a[:, :-k]], axis=1)


def _router_body(x_ref, gw_ref, w_ref, dA_ref, dB_ref, tm_ref):
    x = x_ref[...]
    logits = lax.dot_general(x, gw_ref[...], (((1,), (1,)), ((), ())),
                             preferred_element_type=jnp.float32)
    mx = jnp.max(logits, axis=-1, keepdims=True)
    ex = jnp.exp(logits - mx)
    scores = ex / jnp.sum(ex, axis=-1, keepdims=True)
    iota = lax.broadcasted_iota(jnp.int32, scores.shape, 1)
    m1 = jnp.max(scores, axis=-1, keepdims=True)
    i1 = jnp.min(jnp.where(scores == m1, iota, E), axis=-1, keepdims=True)
    sel1 = iota == i1
    masked = jnp.where(sel1, -jnp.inf, scores)
    m2 = jnp.max(masked, axis=-1, keepdims=True)
    i2 = jnp.min(jnp.where(masked == m2, iota, E), axis=-1, keepdims=True)
    sel2 = iota == i2
    maskf = jnp.where(sel1 | sel2, 1.0, 0.0).astype(jnp.float32)

    acc = _shift0(maskf, 1)
    k = 1
    while k < N:
        acc = acc + _shift0(acc, k)
        k *= 2
    rank = acc
    counts = rank[N - 1:N, :] + maskf[N - 1:N, :]
    tiles = jnp.floor((counts + (GT - 1)) / GT)
    texc = _shift1(tiles, 1)
    texc = texc + _shift1(texc, 1)
    texc = texc + _shift1(texc, 2)
    texc = texc + _shift1(texc, 4)
    seg_off = GT * texc
    cum_incl = texc + tiles

    pos = seg_off + rank
    dA_ref[...] = jnp.sum(jnp.where(sel1, pos, 0.0), axis=1,
                          keepdims=True).astype(jnp.int32)
    dB_ref[...] = jnp.sum(jnp.where(sel2, pos, 0.0), axis=1,
                          keepdims=True).astype(jnp.int32)
    w_ref[...] = jnp.concatenate([m1, m2], axis=1)

    tio = lax.broadcasted_iota(jnp.int32, (1, 64), 1).astype(jnp.float32)
    tm = jnp.zeros((1, 64), jnp.float32)
    for e in range(E):
        tm = tm + jnp.where(tio >= cum_incl[:, e:e + 1], 1.0, 0.0)
    tm_ref[...] = tm.astype(jnp.int32)


def _gffn_body(tm_ref, xs_ref, gu_ref, dn_ref, ys_ref):
    t = pl.program_id(0)

    @pl.when(tm_ref[t] < E)
    def _():
        xb = xs_ref[...]
        gu = lax.dot_general(xb, gu_ref[0], (((1,), (1,)), ((), ())),
                             preferred_element_type=jnp.float32)
        g = gu[:, :I]
        u = gu[:, I:]
        h = g * jax.nn.sigmoid(g) * u
        ys_ref[...] = lax.dot_general(h, dn_ref[0], (((1,), (1,)), ((), ())),
                                      preferred_element_type=jnp.float32)


def _shared_body(x_ref, sg_ref, su_ref, sd_ref, seg_ref, out_ref):
    s2 = pl.program_id(0)
    t = pl.program_id(1)
    tb = pl.ds(t * BT, BT)
    xb = x_ref[...]
    g = lax.dot_general(xb, sg_ref[...], (((1,), (1,)), ((), ())),
                        preferred_element_type=jnp.float32)
    u = lax.dot_general(xb, su_ref[...], (((1,), (1,)), ((), ())),
                        preferred_element_type=jnp.float32)
    h = g * jax.nn.sigmoid(g) * u
    sh = lax.dot_general(h, sd_ref[...], (((1,), (1,)), ((), ())),
                         preferred_element_type=jnp.float32)

    @pl.when(s2 == 0)
    def _():
        out_ref[tb, :] = sh

    @pl.when(s2 == 1)
    def _():
        gate_logit = jnp.sum(xb * seg_ref[...], axis=1, keepdims=True)
        out_ref[tb, :] = (out_ref[tb, :] + sh) * jax.nn.sigmoid(gate_logit)


def _combine_body(y0_ref, y1_ref, w_ref, sh_ref, out_ref):
    w = w_ref[...]
    out_ref[...] = (w[:, 0:1] * y0_ref[...] + w[:, 1:2] * y1_ref[...]
                    + sh_ref[...])


@functools.cache
def _sc_mesh():
    return plsc.VectorSubcoreMesh(core_axis_name="c", subcore_axis_name="s",
                                  num_cores=NC, num_subcores=NSUB)


def _dispatch_tokens(xf, dA, dB):
    @functools.partial(
        pl.kernel,
        out_type=jax.ShapeDtypeStruct((NS, H), jnp.float32),
        mesh=_sc_mesh(),
        scratch_types=[
            pltpu.VMEM((BPW,), jnp.int32),
            pltpu.VMEM((BPW,), jnp.int32),
            pltpu.VMEM((BPW, H), jnp.float32),
        ],
    )
    def scatter_kernel(x_hbm, dA_hbm, dB_hbm, xs_hbm, idxA_v, idxB_v, rows_v):
        wid = lax.axis_index("s") * NC + lax.axis_index("c")
        base = wid * BPW
        pltpu.sync_copy(dA_hbm.at[pl.ds(base, BPW)], idxA_v)
        pltpu.sync_copy(dB_hbm.at[pl.ds(base, BPW)], idxB_v)
        pltpu.sync_copy(x_hbm.at[pl.ds(base, BPW)], rows_v)
        pltpu.sync_copy(rows_v, xs_hbm.at[idxA_v])
        pltpu.sync_copy(rows_v, xs_hbm.at[idxB_v])

    return scatter_kernel(xf, dA, dB)


def _collect_rows(ys, dA, dB):
    @functools.partial(
        pl.kernel,
        out_type=(jax.ShapeDtypeStruct((N, H), jnp.float32),
                  jax.ShapeDtypeStruct((N, H), jnp.float32)),
        mesh=_sc_mesh(),
        scratch_types=[
            pltpu.VMEM((BPW,), jnp.int32),
            pltpu.VMEM((BPW, H), jnp.float32),
            pltpu.SemaphoreType.DMA,
        ],
    )
    def gather_kernel(ys_hbm, dA_hbm, dB_hbm, y0_hbm, y1_hbm, idx_v, rows_v, sem):
        wid = lax.axis_index("s") * NC + lax.axis_index("c")
        base = wid * BPW
        pltpu.sync_copy(dA_hbm.at[pl.ds(base, BPW)], idx_v)
        pltpu.async_copy(ys_hbm.at[idx_v], rows_v, sem).wait()
        pltpu.sync_copy(rows_v, y0_hbm.at[pl.ds(base, BPW)])
        pltpu.sync_copy(dB_hbm.at[pl.ds(base, BPW)], idx_v)
        pltpu.async_copy(ys_hbm.at[idx_v], rows_v, sem).wait()
        pltpu.sync_copy(rows_v, y1_hbm.at[pl.ds(base, BPW)])

    return gather_kernel(ys, dA, dB)


def kernel(x, gate_w, experts_gate_up, experts_down, shared_gate_w,
           shared_up_w, shared_down_w, shared_expert_gate_w):
    Bx, Tx, Hx = x.shape
    xf = x.reshape(Bx * Tx, Hx)

    w2, dA2, dB2, tm = pl.pallas_call(
        _router_body,
        grid=(1,),
        in_specs=[
            pl.BlockSpec((N, H), lambda i: (0, 0)),
            pl.BlockSpec((E, H), lambda i: (0, 0)),
        ],
        out_specs=[
            pl.BlockSpec((N, 2), lambda i: (0, 0)),
            pl.BlockSpec((N, 1), lambda i: (0, 0)),
            pl.BlockSpec((N, 1), lambda i: (0, 0)),
            pl.BlockSpec((1, 64), lambda i: (0, 0)),
        ],
        out_shape=[
            jax.ShapeDtypeStruct((N, 2), jnp.float32),
            jax.ShapeDtypeStruct((N, 1), jnp.int32),
            jax.ShapeDtypeStruct((N, 1), jnp.int32),
            jax.ShapeDtypeStruct((1, 64), jnp.int32),
        ],
    )(xf, gate_w)
    dA = dA2.reshape(N)
    dB = dB2.reshape(N)
    tile_map = tm.reshape(64)[:NTILES]

    sh = pl.pallas_call(
        _shared_body,
        grid=(2, N // BT),
        in_specs=[
            pl.BlockSpec((BT, H), lambda s2, t: (t, 0)),
            pl.BlockSpec((S // 2, H), lambda s2, t: (s2, 0)),
            pl.BlockSpec((S // 2, H), lambda s2, t: (s2, 0)),
            pl.BlockSpec((H, S // 2), lambda s2, t: (0, s2)),
            pl.BlockSpec((1, H), lambda s2, t: (0, 0)),
        ],
        out_specs=pl.BlockSpec((N, H), lambda s2, t: (0, 0)),
        out_shape=jax.ShapeDtypeStruct((N, H), jnp.float32),
    )(xf, shared_gate_w, shared_up_w, shared_down_w, shared_expert_gate_w)

    xsf = _dispatch_tokens(xf, dA, dB)

    ys = pl.pallas_call(
        _gffn_body,
        grid_spec=pltpu.PrefetchScalarGridSpec(
            num_scalar_prefetch=1,
            grid=(NTILES,),
            in_specs=[
                pl.BlockSpec((GT, H), lambda t, tm_ref: (t, 0)),
                pl.BlockSpec((1, 2 * I, H),
                             lambda t, tm_ref: (jnp.minimum(tm_ref[t], E - 1), 0, 0)),
                pl.BlockSpec((1, H, I),
                             lambda t, tm_ref: (jnp.minimum(tm_ref[t], E - 1), 0, 0)),
            ],
            out_specs=pl.BlockSpec((GT, H), lambda t, tm_ref: (t, 0)),
        ),
        out_shape=jax.ShapeDtypeStruct((NS, H), jnp.float32),
    )(tile_map, xsf, experts_gate_up, experts_down)

    y0, y1 = _collect_rows(ys, dA, dB)

    out = pl.pallas_call(
        _combine_body,
        grid=(N // BT,),
        in_specs=[
            pl.BlockSpec((BT, H), lambda t: (t, 0)),
            pl.BlockSpec((BT, H), lambda t: (t, 0)),
            pl.BlockSpec((BT, 2), lambda t: (t, 0)),
            pl.BlockSpec((BT, H), lambda t: (t, 0)),
        ],
        out_specs=pl.BlockSpec((BT, H), lambda t: (t, 0)),
        out_shape=jax.ShapeDtypeStruct((N, H), jnp.float32),
    )(y0, y1, w2, sh)

    return out.reshape(Bx, Tx, Hx)
